# baseline (speedup 1.0000x reference)
import jax
import jax.numpy as jnp
from jax.experimental import pallas as pl

N = 10000
H = 8
D = 16


def _div_body(wv_ref, z_ref, o_ref):
    o_ref[...] = wv_ref[...] / z_ref[...]


_div = pl.pallas_call(
    _div_body,
    grid=(10,),
    in_specs=[pl.BlockSpec((1000, H, D), lambda i: (i, 0, 0))] * 2,
    out_specs=pl.BlockSpec((1000, H, D), lambda i: (i, 0, 0)),
    out_shape=jax.ShapeDtypeStruct((N, H, D), jnp.float32),
)


def _proj_body(h_ref, wq_ref, wk_ref, wv_ref, q_ref, k_ref, v_ref):
    hb = h_ref[...]
    q_ref[...] = jnp.dot(hb, wq_ref[...], preferred_element_type=jnp.float32)
    k_ref[...] = jnp.dot(hb, wk_ref[...], preferred_element_type=jnp.float32)
    v_ref[...] = jnp.dot(hb, wv_ref[...], preferred_element_type=jnp.float32)


_proj = pl.pallas_call(
    _proj_body,
    grid=(10,),
    in_specs=[
        pl.BlockSpec((1000, 128), lambda i: (i, 0)),
        pl.BlockSpec((128, 128), lambda i: (0, 0)),
        pl.BlockSpec((128, 128), lambda i: (0, 0)),
        pl.BlockSpec((128, 128), lambda i: (0, 0)),
    ],
    out_specs=[pl.BlockSpec((1000, 128), lambda i: (i, 0))] * 3,
    out_shape=[jax.ShapeDtypeStruct((N, 128), jnp.float32)] * 3,
)


def kernel(h, edge_index, W_Q, W_K, W_V):
    Q_f, K_f, V_f = _proj(h, W_Q, W_K, W_V)
    Q_h = Q_f.reshape(-1, H, D)
    K_h = K_f.reshape(-1, H, D)
    V_h = V_f.reshape(-1, H, D)
    src = edge_index[0]
    dst = edge_index[1]
    score = K_h[src] * Q_h[dst]
    score = score / jnp.sqrt(jnp.float32(D))
    order = jnp.argsort(dst, stable=True)
    dst_s = dst[order]
    score_s = score[order]
    upd_s = (V_h[src] * score)[order]
    wV = jnp.zeros((N, H, D), jnp.float32).at[dst_s].add(upd_s)
    zden = jnp.full((N, H, D), 1e-06, jnp.float32).at[dst_s].add(score_s)
    out = _div(wV, zden)
    return out


# speedup vs baseline: 63.6346x; 63.6346x over previous
import functools

import jax
import jax.numpy as jnp
from jax import lax
from jax.experimental import pallas as pl
from jax.experimental.pallas import tpu as pltpu
from jax.experimental.pallas import tpu_sc as plsc

N = 10000
E = 160000
IN_DIM = 128
H = 8
D = 16
HD = H * D

N_PAD = 10240
BLK = 1024

NR = 160
NRANGE = 64
CAP = 3456
ECH = 2000
NSCAN = E // ECH
GCH = 128


def _proj_body(h_ref, wq_ref, wk_ref, wv_ref, q_ref, kv_ref):
    hb = h_ref[...]
    q_ref[...] = jnp.dot(hb, wq_ref[...], preferred_element_type=jnp.float32)
    kv_ref[:, :HD] = jnp.dot(hb, wk_ref[...], preferred_element_type=jnp.float32)
    kv_ref[:, HD:] = jnp.dot(hb, wv_ref[...], preferred_element_type=jnp.float32)


_proj = pl.pallas_call(
    _proj_body,
    grid=(N_PAD // BLK,),
    in_specs=[
        pl.BlockSpec((BLK, IN_DIM), lambda i: (i, 0)),
        pl.BlockSpec((IN_DIM, HD), lambda i: (0, 0)),
        pl.BlockSpec((IN_DIM, HD), lambda i: (0, 0)),
        pl.BlockSpec((IN_DIM, HD), lambda i: (0, 0)),
    ],
    out_specs=[
        pl.BlockSpec((BLK, HD), lambda i: (i, 0)),
        pl.BlockSpec((BLK, 2 * HD), lambda i: (i, 0)),
    ],
    out_shape=[
        jax.ShapeDtypeStruct((N_PAD, HD), jnp.float32),
        jax.ShapeDtypeStruct((N_PAD, 2 * HD), jnp.float32),
    ],
)


def _div_body(wv_ref, z_ref, o_ref):
    o_ref[...] = wv_ref[...] / z_ref[...]


_div = pl.pallas_call(
    _div_body,
    grid=(10,),
    in_specs=[pl.BlockSpec((1000, HD), lambda i: (i, 0))] * 2,
    out_specs=pl.BlockSpec((1000, HD), lambda i: (i, 0)),
    out_shape=jax.ShapeDtypeStruct((N, HD), jnp.float32),
)


_mesh = plsc.VectorSubcoreMesh(core_axis_name="c", subcore_axis_name="s")

_ONES16 = None


@functools.partial(
    pl.kernel,
    mesh=_mesh,
    compiler_params=pltpu.CompilerParams(needs_layout_passes=False),
    out_type=[
        jax.ShapeDtypeStruct((N, HD), jnp.float32),
        jax.ShapeDtypeStruct((N, HD), jnp.float32),
    ],
    scratch_types=[
        pltpu.VMEM((168, HD), jnp.float32),
        pltpu.VMEM((168, HD), jnp.float32),
        pltpu.VMEM((168, HD), jnp.float32),
        pltpu.VMEM((GCH, 2 * HD), jnp.float32),
        pltpu.VMEM((ECH,), jnp.int32),
        pltpu.VMEM((ECH,), jnp.int32),
        pltpu.VMEM((CAP,), jnp.int32),
        pltpu.VMEM((CAP,), jnp.int32),
        pltpu.VMEM((CAP,), jnp.int32),
        pltpu.VMEM((CAP,), jnp.int32),
        pltpu.SemaphoreType.DMA,
    ],
)
def _sc_attn(kv_hbm, q_hbm, src_hbm, dst_hbm,
             wv_out, z_out,
             acc_wv, acc_z, q_local, rows,
             src_st, dst_st, sl0, dl0, sl1, dl1, sem):
    c = lax.axis_index("c")
    s = lax.axis_index("s")
    w = s * 2 + c
    lo0 = w * NR
    lo1 = (w + 32) * NR

    zeros16 = jnp.zeros((16,), jnp.int32)
    ones16 = jnp.full((16,), 1, jnp.int32)
    pad_d16 = jnp.full((16,), NR, jnp.int32)

    def prefill(i, carry):
        sl0[pl.ds(i * 16, 16)] = zeros16
        dl0[pl.ds(i * 16, 16)] = pad_d16
        sl1[pl.ds(i * 16, 16)] = zeros16
        dl1[pl.ds(i * 16, 16)] = pad_d16
        return carry
    lax.fori_loop(0, CAP // 16, prefill, 0)

    def scan_chunk(i, tails):
        pltpu.sync_copy(src_hbm.at[pl.ds(i * ECH, ECH)], src_st)
        pltpu.sync_copy(dst_hbm.at[pl.ds(i * ECH, ECH)], dst_st)

        def group(jg, tails):
            t0, t1 = tails
            src16 = src_st[pl.ds(jg * 16, 16)]
            dst16 = dst_st[pl.ds(jg * 16, 16)]

            m0 = (dst16 >= lo0) & (dst16 < lo0 + NR)
            m0i = jnp.where(m0, ones16, zeros16)
            pos0 = t0 + plsc.cumsum(m0i) - m0i
            m0w = m0 & (pos0 < CAP)
            plsc.store_scatter(sl0, [pos0], src16, mask=m0w)
            plsc.store_scatter(dl0, [pos0], dst16 - lo0, mask=m0w)
            t0 = t0 + plsc.all_reduce_population_count(m0w)[0]

            m1 = (dst16 >= lo1) & (dst16 < lo1 + NR)
            m1i = jnp.where(m1, ones16, zeros16)
            pos1 = t1 + plsc.cumsum(m1i) - m1i
            m1w = m1 & (pos1 < CAP)
            plsc.store_scatter(sl1, [pos1], src16, mask=m1w)
            plsc.store_scatter(dl1, [pos1], dst16 - lo1, mask=m1w)
            t1 = t1 + plsc.all_reduce_population_count(m1w)[0]
            return (t0, t1)

        return lax.fori_loop(0, ECH // 16, group, tails)

    t0, t1 = lax.fori_loop(0, NSCAN, scan_chunk, (0, 0))

    zero16f = jnp.zeros((16,), jnp.float32)
    eps16f = jnp.full((16,), 1e-06, jnp.float32)

    for r, (lo, tail, slist, dlist) in enumerate(
            ((lo0, t0, sl0, dl0), (lo1, t1, sl1, dl1))):
        g = w + 32 * r

        def initacc(i, carry):
            for j in range(8):
                acc_wv[i, pl.ds(j * 16, 16)] = zero16f
                acc_z[i, pl.ds(j * 16, 16)] = eps16f
            return carry
        lax.fori_loop(0, 168, initacc, 0)

        pltpu.sync_copy(q_hbm.at[pl.ds(lo, NR)], q_local.at[pl.ds(0, NR)])

        def zrow(i, carry):
            for j in range(8):
                q_local[NR + i, pl.ds(j * 16, 16)] = zero16f
            return carry
        lax.fori_loop(0, 8, zrow, 0)

        nch = (tail + (GCH - 1)) // GCH

        def chunk(ch, carry):
            pltpu.async_copy(
                kv_hbm.at[slist.at[pl.ds(ch * GCH, GCH)]], rows, sem).wait()

            def group16(i, carry2):
                d16 = dlist[pl.ds(ch * GCH + i * 16, 16)]
                for k in range(16):
                    d = d16[k]
                    e = i * 16 + k
                    for j in range(8):
                        cs = pl.ds(j * 16, 16)
                        kj = rows[e, cs]
                        vj = rows[e, pl.ds(HD + j * 16, 16)]
                        sc = (kj * q_local[d, cs]) * jnp.float32(0.25)
                        acc_z[d, cs] += sc
                        acc_wv[d, cs] += vj * sc
                return carry2
            lax.fori_loop(0, GCH // 16, group16, 0)
            return carry
        lax.fori_loop(0, nch, chunk, 0)

        @pl.when(g < 62)
        def _():
            pltpu.sync_copy(acc_wv.at[pl.ds(0, NR)], wv_out.at[pl.ds(lo, NR)])
            pltpu.sync_copy(acc_z.at[pl.ds(0, NR)], z_out.at[pl.ds(lo, NR)])

        @pl.when(g == 62)
        def _():
            pltpu.sync_copy(acc_wv.at[pl.ds(0, 80)], wv_out.at[pl.ds(lo, 80)])
            pltpu.sync_copy(acc_z.at[pl.ds(0, 80)], z_out.at[pl.ds(lo, 80)])


def kernel(h, edge_index, W_Q, W_K, W_V):
    h_pad = jnp.concatenate(
        [h, jnp.zeros((N_PAD - N, IN_DIM), jnp.float32)], axis=0)
    q, kv = _proj(h_pad, W_Q, W_K, W_V)

    src = edge_index[0].astype(jnp.int32)
    dst = edge_index[1].astype(jnp.int32)

    wv, zden = _sc_attn(kv, q, src, dst)
    out = _div(wv, zden)
    return out.reshape(N, H, D)
